# baseline (speedup 1.0000x reference)
import functools

import jax
import jax.numpy as jnp
from jax import lax
from jax.experimental import pallas as pl
from jax.experimental.pallas import tpu as pltpu
from jax.experimental.pallas import tpu_sc as plsc

_PAD = 1
_B, _S, _D = 4, 2048, 1024
_NC, _NS, _L = 2, 16, 16
_NW = _NC * _NS
_TPW = (_B * _S) // _NW
_WPR = _NW // _B
_G = 16
_NCH = _TPW // _G
_DT = 2
_DP = 4

_mesh = plsc.VectorSubcoreMesh(core_axis_name="c", subcore_axis_name="s")


@functools.partial(
    pl.kernel,
    out_type=(
        jax.ShapeDtypeStruct((_B, _S, _D), jnp.float32),
        jax.ShapeDtypeStruct((_B, _S), jnp.int32),
    ),
    mesh=_mesh,
    scratch_types=[
        pltpu.VMEM((_S,), jnp.int32),
        pltpu.VMEM((_TPW,), jnp.int32),
        pltpu.VMEM((_TPW,), jnp.int32),
        pltpu.VMEM((_DT, _G, _D), jnp.float32),
        pltpu.VMEM((_DP, _G, _D), jnp.float32),
        pltpu.SemaphoreType.DMA((_DT,)),
        pltpu.SemaphoreType.DMA((_DP,)),
        pltpu.SemaphoreType.DMA((_DP,)),
    ],
    compiler_params=pltpu.CompilerParams(needs_layout_passes=False),
)
def _embed_kernel(ids_hbm, tok_hbm, pos_hbm, out_hbm, attn_hbm,
                  row_v, pid_v, attn_v, tokb, posb, semt, semp, semo):
    wid = lax.axis_index("s") * _NC + lax.axis_index("c")
    brow = wid // _WPR
    o = (wid % _WPR) * _TPW

    pltpu.sync_copy(ids_hbm.at[brow], row_v)

    def _tok_copy(ch, bt):
        return pltpu.make_async_copy(
            tok_hbm.at[row_v.at[pl.ds(o + ch * _G, _G)]], tokb.at[bt],
            semt.at[bt])

    def _pos_copy(ch, bp):
        return pltpu.make_async_copy(
            pos_hbm.at[pid_v.at[pl.ds(ch * _G, _G)]], posb.at[bp],
            semp.at[bp])

    def _out_copy(ch, bp):
        return pltpu.make_async_copy(
            posb.at[bp],
            out_hbm.at[brow, pl.ds(o + ch * _G, _G), :],
            semo.at[bp])

    _tok_copy(0, 0).start()
    _tok_copy(1, 1).start()

    def _pc_body(i, acc):
        ids16 = row_v[pl.ds(i * _L, _L)]
        return acc + jnp.where(ids16 != _PAD, 1, 0)

    acc = lax.fori_loop(0, o // _L, _pc_body, jnp.zeros((_L,), jnp.int32))
    carry0 = jnp.sum(acc)

    def _pos_body(k, carry):
        ids16 = row_v[pl.ds(o + k * _L, _L)]
        attn16 = jnp.where(ids16 != _PAD, 1, 0)
        cum = plsc.cumsum(attn16)
        pos16 = jnp.maximum(carry + cum - 1, 0)
        pid_v[pl.ds(k * _L, _L)] = pos16
        attn_v[pl.ds(k * _L, _L)] = attn16
        return carry + jnp.sum(attn16)

    lax.fori_loop(0, _TPW // _L, _pos_body, carry0)

    _pos_copy(0, 0).start()
    _pos_copy(1, 1).start()
    pltpu.sync_copy(attn_v, attn_hbm.at[brow, pl.ds(o, _TPW)])

    _DH = _D // 2

    def _accum_half(bt, bp, h):
        tr, pr = tokb.at[bt], posb.at[bp]

        @plsc.parallel_loop(h * (_DH // _L), (h + 1) * (_DH // _L))
        def _(j):
            for r in range(_G):
                x = tr[r, pl.ds(j * _L, _L)]
                plsc.addupdate(pr.at[r, pl.ds(j * _L, _L)], x)

    def _out_half(ch, bp, h):
        return pltpu.make_async_copy(
            posb.at[bp, slice(None), pl.ds(h * _DH, _DH)],
            out_hbm.at[brow, pl.ds(o + ch * _G, _G), pl.ds(h * _DH, _DH)],
            semo.at[bp])

    def _accum_out(ch, bt, bp):
        _accum_half(bt, bp, 0)
        _out_half(ch, bp, 0).start()
        _accum_half(bt, bp, 1)
        _out_half(ch, bp, 1).start()

    def _core(g, _):
        for b in range(_DP):
            ch = g * _DP + b
            bt, bp = b % _DT, b % _DP
            nbp = (b + 2) % _DP
            _tok_copy(ch, bt).wait()
            _pos_copy(ch, bp).wait()
            if b < 2:
                @pl.when(ch >= 2)
                def _():
                    _out_copy(ch - 2, nbp).wait()
                _pos_copy(ch + 2, nbp).start()
                _accum_out(ch, bt, bp)
                _tok_copy(ch + 2, (b + 2) % _DT).start()
            else:
                _out_copy(ch - 2, nbp).wait()

                @pl.when(ch + 2 < _NCH)
                def _():
                    _pos_copy(ch + 2, nbp).start()
                _accum_out(ch, bt, bp)

                @pl.when(ch + 2 < _NCH)
                def _():
                    _tok_copy(ch + 2, (b + 2) % _DT).start()
        return 0

    lax.fori_loop(0, _NCH // _DP, _core, 0)

    for ch in range(_NCH - 2, _NCH):
        _out_copy(ch, ch % _DP).wait()


def kernel(input_ids, embed_tokens, embed_positions):
    return _embed_kernel(input_ids, embed_tokens, embed_positions)


# speedup vs baseline: 1.0256x; 1.0256x over previous
import functools

import jax
import jax.numpy as jnp
from jax import lax
from jax.experimental import pallas as pl
from jax.experimental.pallas import tpu as pltpu
from jax.experimental.pallas import tpu_sc as plsc

_PAD = 1
_B, _S, _D = 4, 2048, 1024
_NC, _NS, _L = 2, 16, 16
_NW = _NC * _NS
_TPW = (_B * _S) // _NW
_WPR = _NW // _B
_G = 16
_NCH = _TPW // _G
_DT = 2
_DP = 4

_mesh = plsc.VectorSubcoreMesh(core_axis_name="c", subcore_axis_name="s")


@functools.partial(
    pl.kernel,
    out_type=(
        jax.ShapeDtypeStruct((_B, _S, _D), jnp.float32),
        jax.ShapeDtypeStruct((_B, _S), jnp.int32),
    ),
    mesh=_mesh,
    scratch_types=[
        pltpu.VMEM((_S,), jnp.int32),
        pltpu.VMEM((_TPW,), jnp.int32),
        pltpu.VMEM((_TPW,), jnp.int32),
        pltpu.VMEM((_DT, _G, _D), jnp.float32),
        pltpu.VMEM((_DP, _G, _D), jnp.float32),
        pltpu.SemaphoreType.DMA((_DT,)),
        pltpu.SemaphoreType.DMA((_DP,)),
        pltpu.SemaphoreType.DMA((_DP,)),
    ],
    compiler_params=pltpu.CompilerParams(needs_layout_passes=False),
)
def _embed_kernel(ids_hbm, tok_hbm, pos_hbm, out_hbm, attn_hbm,
                  row_v, pid_v, attn_v, tokb, posb, semt, semp, semo):
    wid = lax.axis_index("s") * _NC + lax.axis_index("c")
    brow = wid // _WPR
    o = (wid % _WPR) * _TPW

    pltpu.sync_copy(ids_hbm.at[brow], row_v)

    def _tok_copy(ch, bt):
        return pltpu.make_async_copy(
            tok_hbm.at[row_v.at[pl.ds(o + ch * _G, _G)]], tokb.at[bt],
            semt.at[bt])

    def _pos_copy(ch, bp):
        return pltpu.make_async_copy(
            pos_hbm.at[pid_v.at[pl.ds(ch * _G, _G)]], posb.at[bp],
            semp.at[bp])

    def _out_copy(ch, bp):
        return pltpu.make_async_copy(
            posb.at[bp],
            out_hbm.at[brow, pl.ds(o + ch * _G, _G), :],
            semo.at[bp])

    _tok_copy(0, 0).start()
    _tok_copy(1, 1).start()

    def _pc_body(i, acc):
        ids16 = row_v[pl.ds(i * _L, _L)]
        return acc + jnp.where(ids16 != _PAD, 1, 0)

    acc = lax.fori_loop(0, o // _L, _pc_body, jnp.zeros((_L,), jnp.int32))
    carry0 = jnp.sum(acc)

    def _pos_body(k, carry):
        ids16 = row_v[pl.ds(o + k * _L, _L)]
        attn16 = jnp.where(ids16 != _PAD, 1, 0)
        cum = plsc.cumsum(attn16)
        pos16 = jnp.maximum(carry + cum - 1, 0)
        pid_v[pl.ds(k * _L, _L)] = pos16
        attn_v[pl.ds(k * _L, _L)] = attn16
        return carry + jnp.sum(attn16)

    lax.fori_loop(0, _TPW // _L, _pos_body, carry0)

    _pos_copy(0, 0).start()
    _pos_copy(1, 1).start()
    pltpu.sync_copy(attn_v, attn_hbm.at[brow, pl.ds(o, _TPW)])

    def _accum_out(ch, bt, bp):
        tr, pr = tokb.at[bt], posb.at[bp]

        @plsc.parallel_loop(0, _D // _L)
        def _(j):
            for r in range(_G):
                x = tr[r, pl.ds(j * _L, _L)]
                plsc.addupdate(pr.at[r, pl.ds(j * _L, _L)], x)

        _out_copy(ch, bp).start()

    def _core(g, _):
        for b in range(_DP):
            ch = g * _DP + b
            bt, bp = b % _DT, b % _DP
            nbp = (b + 2) % _DP
            _tok_copy(ch, bt).wait()
            _pos_copy(ch, bp).wait()
            if b < 2:
                @pl.when(ch >= 2)
                def _():
                    _out_copy(ch - 2, nbp).wait()
                _pos_copy(ch + 2, nbp).start()
                _accum_out(ch, bt, bp)
                _tok_copy(ch + 2, (b + 2) % _DT).start()
            else:
                _out_copy(ch - 2, nbp).wait()

                @pl.when(ch + 2 < _NCH)
                def _():
                    _pos_copy(ch + 2, nbp).start()
                _accum_out(ch, bt, bp)

                @pl.when(ch + 2 < _NCH)
                def _():
                    _tok_copy(ch + 2, (b + 2) % _DT).start()
        return 0

    lax.fori_loop(0, _NCH // _DP, _core, 0)

    for ch in range(_NCH - 2, _NCH):
        _out_copy(ch, ch % _DP).wait()


def kernel(input_ids, embed_tokens, embed_positions):
    return _embed_kernel(input_ids, embed_tokens, embed_positions)
